# baseline (speedup 1.0000x reference)
import functools

import jax
import jax.numpy as jnp
from jax import lax
from jax.experimental import pallas as pl
from jax.experimental.pallas import tpu as pltpu
from jax.experimental.pallas import tpu_sc as plsc

B = 16384
F = 26
V = 100000
D = 16
H1 = 16
H2 = 32
EPS = 1e-5

total = F * V
BF = B * F
NW = 32
ROWS_W = BF // NW
G = 128
NG = ROWS_W // G
NCHUNK = 8
GPC = NG // NCHUNK
CROWS = GPC * G


def _sc_gather(idx2d, emb_w, lin_w):
  mesh = plsc.VectorSubcoreMesh(core_axis_name="c", subcore_axis_name="s")

  @functools.partial(
      pl.kernel,
      out_type=[
          jax.ShapeDtypeStruct((BF, D), jnp.float32),
          jax.ShapeDtypeStruct((BF,), jnp.float32),
      ],
      mesh=mesh,
      compiler_params=pltpu.CompilerParams(use_tc_tiling_on_sc=False),
      scratch_types=[
          pltpu.VMEM((NG, G), jnp.int32),
          pltpu.VMEM((2 * CROWS, D), jnp.float32),
          pltpu.VMEM((2 * CROWS,), jnp.float32),
          pltpu.SemaphoreType.DMA,
      ],
  )
  def body(idx_hbm, emb_hbm, lin_hbm, e_out, linv_out, idx_v, rows_v, lrows_v,
           sem):
    wid = lax.axis_index("s") * 2 + lax.axis_index("c")
    wbase = wid * ROWS_W
    pltpu.sync_copy(idx_hbm.at[pl.ds(wid * NG, NG), :], idx_v)

    def fire(c, buf):
      copies = []
      for j in range(GPC):
        irow = idx_v.at[c * GPC + j]
        dst = pl.ds(buf * CROWS + j * G, G)
        copies.append(
            pltpu.async_copy(emb_hbm.at[irow], rows_v.at[dst, :], sem))
        copies.append(
            pltpu.async_copy(lin_hbm.at[irow], lrows_v.at[dst], sem))
      return copies

    def drain(copies):
      for cp in copies:
        cp.wait()

    def writeout(c, buf):
      src = pl.ds(buf * CROWS, CROWS)
      out_rows = pl.ds(wbase + c * CROWS, CROWS)
      pltpu.sync_copy(rows_v.at[src, :], e_out.at[out_rows, :])
      pltpu.sync_copy(lrows_v.at[src], linv_out.at[out_rows])

    inflight = fire(0, 0)
    for c in range(NCHUNK):
      drain(inflight)
      if c + 1 < NCHUNK:
        nxt = fire(c + 1, (c + 1) % 2)
      writeout(c, c % 2)
      if c + 1 < NCHUNK:
        inflight = nxt

  return body(idx2d, emb_w, lin_w)


def _tc_body(e_ref, linv_ref, s_ref, w1_ref, b1_ref, w2_ref, b2_ref, w3_ref,
             c0_ref, out_ref):
  e = e_ref[...]
  s = jnp.dot(e, s_ref[...], preferred_element_type=jnp.float32)
  sumsq = jnp.sum(e * e, axis=1, keepdims=True)
  second = 0.5 * (jnp.sum(s * s, axis=1, keepdims=True) - sumsq)
  h = jnp.dot(e, w1_ref[...], preferred_element_type=jnp.float32) + b1_ref[...]
  h = jnp.maximum(h, 0.0)
  h = jnp.dot(h, w2_ref[...], preferred_element_type=jnp.float32) + b2_ref[...]
  h = jnp.maximum(h, 0.0)
  deep = jnp.dot(h, w3_ref[...], preferred_element_type=jnp.float32)
  first = jnp.sum(linv_ref[...], axis=1, keepdims=True)
  out_ref[...] = first + second + deep + c0_ref[0, 0]


def _tc_fused(e, linv, sel, w1f, b1f, w2f, b2f, w3, c0):
  bm = 2048
  grid = (B // bm,)
  return pl.pallas_call(
      _tc_body,
      grid=grid,
      in_specs=[
          pl.BlockSpec((bm, F * D), lambda i: (i, 0)),
          pl.BlockSpec((bm, F), lambda i: (i, 0)),
          pl.BlockSpec((F * D, D), lambda i: (0, 0)),
          pl.BlockSpec((F * D, H1), lambda i: (0, 0)),
          pl.BlockSpec((1, H1), lambda i: (0, 0)),
          pl.BlockSpec((H1, H2), lambda i: (0, 0)),
          pl.BlockSpec((1, H2), lambda i: (0, 0)),
          pl.BlockSpec((H2, 1), lambda i: (0, 0)),
          pl.BlockSpec((1, 1), lambda i: (0, 0)),
      ],
      out_specs=pl.BlockSpec((bm, 1), lambda i: (i, 0)),
      out_shape=jax.ShapeDtypeStruct((B, 1), jnp.float32),
  )(e, linv, sel, w1f, b1f, w2f, b2f, w3, c0)


def kernel(x, emb_w, lin_w, lin_b, W1, b1, g1, be1, rm1, rv1, W2, b2, g2, be2,
           rm2, rv2, W3, b3):
  offsets = (jnp.arange(F, dtype=x.dtype) * V)[None, :]
  idx2d = (x + offsets).reshape(BF // G, G)

  emb2d = emb_w.reshape(total * D).reshape(total, D)
  e_flat, linv_flat = _sc_gather(idx2d, emb2d, lin_w.reshape(total))
  e = e_flat.reshape(B, F * D)
  linv = linv_flat.reshape(B, F)

  inv1 = g1 / jnp.sqrt(rv1 + EPS)
  w1f = W1 * inv1[None, :]
  b1f = ((b1 - rm1) * inv1 + be1)[None, :]
  inv2 = g2 / jnp.sqrt(rv2 + EPS)
  w2f = W2 * inv2[None, :]
  b2f = ((b2 - rm2) * inv2 + be2)[None, :]
  sel = jnp.tile(jnp.eye(D, dtype=jnp.float32), (F, 1))
  c0 = (lin_b + b3).reshape(1, 1)

  out = _tc_fused(e, linv, sel, w1f, b1f, w2f, b2f, W3, c0)
  return e_flat[:B, 0] + linv_flat[:B]


# speedup vs baseline: 1.0908x; 1.0908x over previous
import functools

import jax
import jax.numpy as jnp
from jax import lax
from jax.experimental import pallas as pl
from jax.experimental.pallas import tpu as pltpu
from jax.experimental.pallas import tpu_sc as plsc

B = 16384
F = 26
V = 100000
D = 16
H1 = 16
H2 = 32
EPS = 1e-5

total = F * V
BF = B * F
NW = 32
ROWS_W = BF // NW
G = 128
NG = ROWS_W // G
NCHUNK = 8
GPC = NG // NCHUNK
CROWS = GPC * G
NP = 4
BW = B // NW
NWIN = BW // G


def _sc_gather(xi2d, dst2d, xiT2d, emb_w, lin_w):
  mesh = plsc.VectorSubcoreMesh(core_axis_name="c", subcore_axis_name="s")

  @functools.partial(
      pl.kernel,
      out_type=[
          jax.ShapeDtypeStruct((NP * B * 8, D), jnp.float32),
          jax.ShapeDtypeStruct((BF,), jnp.float32),
      ],
      mesh=mesh,
      compiler_params=pltpu.CompilerParams(use_tc_tiling_on_sc=False),
      scratch_types=[
          pltpu.VMEM((NG, G), jnp.int32),
          pltpu.VMEM((NG, G), jnp.int32),
          pltpu.VMEM((NG, G), jnp.int32),
          pltpu.VMEM((CROWS, D), jnp.float32),
          pltpu.VMEM((CROWS,), jnp.float32),
          pltpu.SemaphoreType.DMA,
          pltpu.SemaphoreType.DMA,
          pltpu.SemaphoreType.DMA,
      ],
  )
  def body(xi_hbm, dst_hbm, xiT_hbm, emb_hbm, lin_hbm, p_out, linvT_out,
           xi_v, dst_v, xiT_v, rows_v, lrows_v, sem, sem_s, sem_l):
    wid = lax.axis_index("s") * 2 + lax.axis_index("c")
    pltpu.sync_copy(xi_hbm.at[pl.ds(wid * NG, NG), :], xi_v)
    pltpu.sync_copy(dst_hbm.at[pl.ds(wid * NG, NG), :], dst_v)
    pltpu.sync_copy(xiT_hbm.at[pl.ds(wid * NG, NG), :], xiT_v)

    @pl.loop(0, NCHUNK)
    def _chunk(c):
      lin_gathers = []
      for j in range(GPC):
        cp = pltpu.async_copy(lin_hbm.at[xiT_v.at[c * GPC + j]],
                              lrows_v.at[pl.ds(j * G, G)], sem_l)
        lin_gathers.append(cp)
      gathers = []
      for j in range(GPC):
        cp = pltpu.async_copy(emb_hbm.at[xi_v.at[c * GPC + j]],
                              rows_v.at[pl.ds(j * G, G), :], sem)
        gathers.append(cp)
      scatters = []
      for j in range(GPC):
        gathers[j].wait()
        cp = pltpu.async_copy(rows_v.at[pl.ds(j * G, G), :],
                              p_out.at[dst_v.at[c * GPC + j]], sem_s)
        scatters.append(cp)
      for cp in lin_gathers:
        cp.wait()
      pltpu.sync_copy(lrows_v,
                      linvT_out.at[pl.ds(wid * ROWS_W + c * CROWS, CROWS)])
      for cp in scatters:
        cp.wait()

  return body(xi2d, dst2d, xiT2d, emb_w, lin_w)


def _tc_body(x0_ref, x1_ref, x2_ref, x3_ref, *rest):
  lin_refs = rest[:F]
  (m3_ref, sel_ref, w1_ref, b1_ref, w2_ref, b2_ref, w3_ref, c0_ref,
   out_ref, first_ref) = rest[F:]
  facc = lin_refs[0][...]
  for r in lin_refs[1:]:
    facc = facc + r[...]
  first_ref[...] = facc
  x0, x1, x2 = x0_ref[...], x1_ref[...], x2_ref[...]
  x3 = x3_ref[...] * m3_ref[...]
  sel = sel_ref[...]
  w1 = w1_ref[...]
  f32 = jnp.float32
  s = (jnp.dot(x0, sel[0:128], preferred_element_type=f32)
       + jnp.dot(x1, sel[128:256], preferred_element_type=f32)
       + jnp.dot(x2, sel[256:384], preferred_element_type=f32)
       + jnp.dot(x3, sel[384:512], preferred_element_type=f32))
  sumsq = (jnp.sum(x0 * x0, axis=1, keepdims=True)
           + jnp.sum(x1 * x1, axis=1, keepdims=True)
           + jnp.sum(x2 * x2, axis=1, keepdims=True)
           + jnp.sum(x3 * x3, axis=1, keepdims=True))
  second = 0.5 * (jnp.sum(s * s, axis=1, keepdims=True) - sumsq)
  h = (jnp.dot(x0, w1[0:128], preferred_element_type=f32)
       + jnp.dot(x1, w1[128:256], preferred_element_type=f32)
       + jnp.dot(x2, w1[256:384], preferred_element_type=f32)
       + jnp.dot(x3, w1[384:512], preferred_element_type=f32)
       + b1_ref[...])
  h = jnp.maximum(h, 0.0)
  h = jnp.dot(h, w2_ref[...], preferred_element_type=f32) + b2_ref[...]
  h = jnp.maximum(h, 0.0)
  deep = jnp.dot(h, w3_ref[...], preferred_element_type=f32)
  out_ref[...] = second + deep + c0_ref[0, 0]


def _tc_fused(p2, lin2d, m3, selp, w1p, b1f, w2f, b2f, w3, c0):
  bm = 2048
  nb = B // bm
  lb = bm // 128
  grid = (nb,)
  xspec = lambda r: pl.BlockSpec((bm, 128), lambda i, r=r: (r * nb + i, 0))
  lspec = lambda f: pl.BlockSpec(
      (lb, 128), lambda i, f=f: (f * (B // 128) // lb + i, 0))
  return pl.pallas_call(
      _tc_body,
      grid=grid,
      in_specs=[
          xspec(0), xspec(1), xspec(2), xspec(3),
          *[lspec(f) for f in range(F)],
          pl.BlockSpec((1, 128), lambda i: (0, 0)),
          pl.BlockSpec((NP * 128, D), lambda i: (0, 0)),
          pl.BlockSpec((NP * 128, H1), lambda i: (0, 0)),
          pl.BlockSpec((1, H1), lambda i: (0, 0)),
          pl.BlockSpec((H1, H2), lambda i: (0, 0)),
          pl.BlockSpec((1, H2), lambda i: (0, 0)),
          pl.BlockSpec((H2, 1), lambda i: (0, 0)),
          pl.BlockSpec((1, 1), lambda i: (0, 0)),
      ],
      out_specs=[
          pl.BlockSpec((bm, 1), lambda i: (i, 0)),
          pl.BlockSpec((lb, 128), lambda i: (i, 0)),
      ],
      out_shape=[
          jax.ShapeDtypeStruct((B, 1), jnp.float32),
          jax.ShapeDtypeStruct((B // 128, 128), jnp.float32),
      ],
  )(p2, p2, p2, p2, *([lin2d] * F), m3, selp, w1p, b1f, w2f, b2f, w3, c0)


def kernel(x, emb_w, lin_w, lin_b, W1, b1, g1, be1, rm1, rv1, W2, b2, g2, be2,
           rm2, rv2, W3, b3):
  offsets = (jnp.arange(F, dtype=x.dtype) * V)[None, :]
  xi = x + offsets
  xi2d = xi.reshape(BF // G, G)
  ar = jnp.arange(BF, dtype=jnp.int32)
  bb, ff = ar // F, ar % F
  dst2d = ((ff // 8) * (B * 8) + bb * 8 + (ff % 8)).reshape(BF // G, G)
  xiT2d = xi.T.reshape(BF // G, G)

  p16, linvT = _sc_gather(xi2d, dst2d, xiT2d,
                          emb_w.reshape(total * D).reshape(total, D),
                          lin_w.reshape(total))
  p2 = p16.reshape(NP * B, 128)
  lin2d = linvT.reshape(BF // G, G)

  inv1 = g1 / jnp.sqrt(rv1 + EPS)
  w1p = jnp.zeros((NP * 128, H1), jnp.float32).at[: F * D].set(
      W1 * inv1[None, :])
  b1f = ((b1 - rm1) * inv1 + be1)[None, :]
  inv2 = g2 / jnp.sqrt(rv2 + EPS)
  w2f = W2 * inv2[None, :]
  b2f = ((b2 - rm2) * inv2 + be2)[None, :]
  selp = jnp.zeros((NP * 128, D), jnp.float32).at[: F * D].set(
      jnp.tile(jnp.eye(D, dtype=jnp.float32), (F, 1)))
  m3 = (jnp.arange(128) < 32).astype(jnp.float32)[None, :]
  c0 = b3.reshape(1, 1)

  out, first = _tc_fused(p2, lin2d, m3, selp, w1p, b1f, w2f, b2f, W3, c0)
  return out.reshape(B) + first.reshape(B) + lin_b[0]
